# baseline (speedup 1.0000x reference)
import functools

import numpy as np
import jax
import jax.numpy as jnp
from jax import lax
from jax.experimental import pallas as pl
from jax.experimental.pallas import tpu as pltpu
from jax.experimental.pallas import tpu_sc as plsc

_NUM_LEVELS = 16
_TBL = 1 << 19
_MASK = _TBL - 1
_P1 = int(np.uint32(2654435761).astype(np.int32))
_P2 = 805459861
_N = 262144
_NC, _NS = 2, 16
_NW = _NC * _NS
_PPW = _N // _NW
_C = 32
_NCHUNK = _PPW // _C
_IPC = _C * 128

_TW = _TBL * _NUM_LEVELS * 2
_WPW = _TW // _NW
_G = 16384
_NGRP = _WPW // _G

_i32 = jnp.int32
_f32 = jnp.float32

_COMPILER_PARAMS = pltpu.CompilerParams(needs_layout_passes=False,
                                        use_tc_tiling_on_sc=False)


def _repack_body(raw, out, in_v, out_v):
  wid = lax.axis_index("s") * _NC + lax.axis_index("c")
  base = wid * _WPW
  iota = lax.iota(_i32, 16)
  pat = (iota >> 1) + ((iota & 1) << 7)

  def group(g, carry):
    gb = base + g * _G
    pltpu.sync_copy(raw.at[pl.ds(gb, _G)], in_v)
    def block(b, c2):
      bb = (b * 256).astype(_i32)
      for i in range(16):
        src = plsc.load_gather(in_v, [bb + 8 * i + pat])
        out_v[pl.ds(bb + 16 * i, 16)] = src
      return c2

    lax.fori_loop(_i32(0), _i32(_G // 256), block, _i32(0))
    pltpu.sync_copy(out_v, out.at[pl.ds(gb, _G)])
    return carry

  lax.fori_loop(_i32(0), _i32(_NGRP), group, _i32(0))


_repack = functools.partial(
    pl.kernel,
    out_type=jax.ShapeDtypeStruct((_TW,), jnp.float32),
    mesh=plsc.VectorSubcoreMesh(core_axis_name="c", subcore_axis_name="s"),
    compiler_params=_COMPILER_PARAMS,
    scratch_types=[
        pltpu.VMEM((_G,), _f32),
        pltpu.VMEM((_G,), _f32),
    ],
)(_repack_body)


def _tec_body(xs, ys, zs, tab, scal, out,
              xs_v, ys_v, zs_v, scal_v,
              idx_v0, sel_v0, off_v0, rows_v0, out_v0, sem0, semo0,
              idx_v1, sel_v1, off_v1, rows_v1, out_v1, sem1, semo1):
  wid = lax.axis_index("s") * _NC + lax.axis_index("c")
  base = wid * _PPW
  pltpu.sync_copy(xs.at[pl.ds(base, _PPW)], xs_v)
  pltpu.sync_copy(ys.at[pl.ds(base, _PPW)], ys_v)
  pltpu.sync_copy(zs.at[pl.ds(base, _PPW)], zs_v)
  pltpu.sync_copy(scal, scal_v)
  scalv = scal_v[...]
  iota = lax.iota(_i32, 16)
  lvloff = iota * _TBL
  bufs = ((idx_v0, sel_v0, off_v0, rows_v0, out_v0, sem0, semo0),
          (idx_v1, sel_v1, off_v1, rows_v1, out_v1, sem1, semo1))

  def build_chunk(g, buf):
    idx_v, sel_v, off_v = buf[0], buf[1], buf[2]
    cb = (g * _C).astype(_i32)

    def build(_, p):
      pidx = jnp.broadcast_to(cb + p, (16,))
      x = plsc.load_gather(xs_v, [pidx])
      y = plsc.load_gather(ys_v, [pidx])
      z = plsc.load_gather(zs_v, [pidx])
      sx = x * scalv
      sy = y * scalv
      sz = z * scalv
      fxi = sx.astype(_i32)
      fyi = sy.astype(_i32)
      fzi = sz.astype(_i32)
      off_v[p, pl.ds(0, 16)] = sx - fxi.astype(_f32)
      off_v[p, pl.ds(16, 16)] = sy - fyi.astype(_f32)
      off_v[p, pl.ds(32, 16)] = sz - fzi.astype(_f32)
      tyf = fyi * _P1
      tzf = fzi * _P2
      tyc = tyf + _P1
      tzc = tzf + _P2
      txf = fxi
      txc = fxi + 1
      acc = txc ^ tyc
      acf = txc ^ tyf
      aff = txf ^ tyf
      afc = txf ^ tyc
      hs = (acc ^ tzc, acf ^ tzc, aff ^ tzc, afc ^ tzc,
            acc ^ tzf, acf ^ tzf, aff ^ tzf, afc ^ tzf)
      pb = p * 128
      for c in range(8):
        h = (hs[c] & _MASK) + lvloff
        idx_v[pl.ds(pb + c * 16, 16)] = h >> 2
        sel_v[pl.ds(pb + c * 16, 16)] = (h << 1) & 6
      return p + 1

    lax.fori_loop(np.int32(0), np.int32(_C), build, _i32(0), unroll=2)
    pltpu.async_copy(tab.at[buf[0]], buf[3], buf[5])

  def drain_chunk(g, buf):
    idx_v, sel_v, off_v, rows_v, out_v, sem, semo = buf
    pltpu.make_async_copy(tab.at[idx_v], rows_v, sem).wait()
    cb = (g * _C).astype(_i32)

    @pl.when(g >= 2)
    def _():
      pltpu.make_async_copy(out_v, out.at[pl.ds(base + cb, _C)], semo).wait()

    def interp(_, p):
      o0 = off_v[p, pl.ds(0, 16)]
      o1 = off_v[p, pl.ds(16, 16)]
      o2 = off_v[p, pl.ds(32, 16)]
      pb = p * 128
      f = []
      for c in range(8):
        rvec = pb + c * 16 + iota
        s0 = sel_v[pl.ds(pb + c * 16, 16)]
        f.append((plsc.load_gather(rows_v, [rvec, s0]),
                  plsc.load_gather(rows_v, [rvec, s0 + 1])))

      def lerp(a, b, o):
        return b + (a - b) * o

      prow = jnp.broadcast_to(p, (16,))
      for k in range(2):
        f03 = lerp(f[0][k], f[3][k], o0)
        f12 = lerp(f[1][k], f[2][k], o0)
        f56 = lerp(f[5][k], f[6][k], o0)
        f47 = lerp(f[4][k], f[7][k], o0)
        f0312 = lerp(f03, f12, o1)
        f4756 = lerp(f47, f56, o1)
        plsc.store_scatter(out_v, [prow, 2 * iota + k], lerp(f0312, f4756, o2))
      return p + 1

    lax.fori_loop(np.int32(0), np.int32(_C), interp, _i32(0), unroll=2)
    pltpu.async_copy(out_v, out.at[pl.ds(base + cb, _C)], semo)

  npair = _NCHUNK // 2
  build_chunk(_i32(0), bufs[0])

  def pair(g2, carry):
    ga = (g2 * 2).astype(_i32)
    build_chunk(ga + 1, bufs[1])
    drain_chunk(ga, bufs[0])

    @pl.when(g2 < npair - 1)
    def _():
      build_chunk(ga + 2, bufs[0])

    drain_chunk(ga + 1, bufs[1])
    return carry

  lax.fori_loop(_i32(0), _i32(npair), pair, _i32(0))
  last = _i32(_NCHUNK - 2).astype(_i32) * _C
  pltpu.make_async_copy(out_v0, out.at[pl.ds(base + last, _C)], semo0).wait()
  pltpu.make_async_copy(out_v1, out.at[pl.ds(base + last, _C)], semo1).wait()


_encode = functools.partial(
    pl.kernel,
    out_type=jax.ShapeDtypeStruct((_N, 2 * _NUM_LEVELS), jnp.float32),
    mesh=plsc.VectorSubcoreMesh(core_axis_name="c", subcore_axis_name="s"),
    compiler_params=_COMPILER_PARAMS,
    scratch_types=[
        pltpu.VMEM((_PPW,), _f32),
        pltpu.VMEM((_PPW,), _f32),
        pltpu.VMEM((_PPW,), _f32),
        pltpu.VMEM((_NUM_LEVELS,), _f32),
        pltpu.VMEM((_IPC,), _i32),
        pltpu.VMEM((_IPC,), _i32),
        pltpu.VMEM((_C, 48), _f32),
        pltpu.VMEM((_IPC, 8), _f32),
        pltpu.VMEM((_C, 2 * _NUM_LEVELS), _f32),
        pltpu.SemaphoreType.DMA,
        pltpu.SemaphoreType.DMA,
        pltpu.VMEM((_IPC,), _i32),
        pltpu.VMEM((_IPC,), _i32),
        pltpu.VMEM((_C, 48), _f32),
        pltpu.VMEM((_IPC, 8), _f32),
        pltpu.VMEM((_C, 2 * _NUM_LEVELS), _f32),
        pltpu.SemaphoreType.DMA,
        pltpu.SemaphoreType.DMA,
    ],
)(_tec_body)


def kernel(in_tensor, hash_table):
  x = in_tensor.astype(jnp.float32)
  raw = (hash_table.astype(jnp.float32)
         .reshape(65536, 128, 2).transpose(0, 2, 1).reshape(_TW))
  tab = _repack(raw).reshape(_TW // 8, 8)
  levels = np.arange(_NUM_LEVELS)
  growth = np.exp((np.log(4096.0) - np.log(16.0)) / (_NUM_LEVELS - 1))
  scal = jnp.asarray(np.floor(16.0 * growth ** levels), dtype=jnp.float32)
  return _encode(x[:, 0], x[:, 1], x[:, 2], tab, scal)


# speedup vs baseline: 1.0390x; 1.0390x over previous
import functools

import numpy as np
import jax
import jax.numpy as jnp
from jax import lax
from jax.experimental import pallas as pl
from jax.experimental.pallas import tpu as pltpu
from jax.experimental.pallas import tpu_sc as plsc

_NUM_LEVELS = 16
_TBL = 1 << 19
_MASK = _TBL - 1
_P1 = int(np.uint32(2654435761).astype(np.int32))
_P2 = 805459861
_N = 262144
_NC, _NS = 2, 16
_NW = _NC * _NS
_PPW = _N // _NW
_C = 32
_NCHUNK = _PPW // _C
_IPC = _C * 128

_TW = _TBL * _NUM_LEVELS * 2
_WPW = _TW // _NW
_G = 16384
_NGRP = _WPW // _G

_i32 = jnp.int32
_f32 = jnp.float32

_COMPILER_PARAMS = pltpu.CompilerParams(needs_layout_passes=False,
                                        use_tc_tiling_on_sc=False)


def _repack_body(raw, out,
                 in_v0, out_v0, semi0, semo0,
                 in_v1, out_v1, semi1, semo1):
  wid = lax.axis_index("s") * _NC + lax.axis_index("c")
  base = wid * _WPW
  iota = lax.iota(_i32, 16)
  pat = (iota >> 1) + ((iota & 1) << 7)
  bufs = ((in_v0, out_v0, semi0, semo0), (in_v1, out_v1, semi1, semo1))

  def start_in(g, buf):
    pltpu.async_copy(raw.at[pl.ds(base + g * _G, _G)], buf[0], buf[2])

  def process(g, buf, fetch_ahead):
    in_v, out_v, semi, semo = buf
    pltpu.make_async_copy(raw.at[pl.ds(base + g * _G, _G)], in_v, semi).wait()

    @pl.when(g >= 2)
    def _():
      pltpu.make_async_copy(out_v, out.at[pl.ds(base + g * _G, _G)],
                            semo).wait()

    def block(_, b):
      bb = b * 256
      for i in range(16):
        src = plsc.load_gather(in_v, [bb + 8 * i + pat])
        out_v[pl.ds(bb + 16 * i, 16)] = src
      return b + 1

    lax.fori_loop(np.int32(0), np.int32(_G // 256), block, _i32(0), unroll=2)
    pltpu.async_copy(out_v, out.at[pl.ds(base + g * _G, _G)], semo)

    @pl.when(fetch_ahead)
    def _():
      start_in(g + 2, buf)

  start_in(_i32(0), bufs[0])
  start_in(_i32(1), bufs[1])
  npair = _NGRP // 2

  def pair(g2, carry):
    ga = (g2 * 2).astype(_i32)
    process(ga, bufs[0], g2 < npair - 1)
    process(ga + 1, bufs[1], g2 < npair - 1)
    return carry

  lax.fori_loop(_i32(0), _i32(npair), pair, _i32(0))
  last = _i32(_NGRP - 2) * _G
  pltpu.make_async_copy(out_v0, out.at[pl.ds(base + last, _G)], semo0).wait()
  pltpu.make_async_copy(out_v1, out.at[pl.ds(base + last, _G)], semo1).wait()


_repack = functools.partial(
    pl.kernel,
    out_type=jax.ShapeDtypeStruct((_TW,), jnp.float32),
    mesh=plsc.VectorSubcoreMesh(core_axis_name="c", subcore_axis_name="s"),
    compiler_params=_COMPILER_PARAMS,
    scratch_types=[
        pltpu.VMEM((_G,), _f32),
        pltpu.VMEM((_G,), _f32),
        pltpu.SemaphoreType.DMA,
        pltpu.SemaphoreType.DMA,
        pltpu.VMEM((_G,), _f32),
        pltpu.VMEM((_G,), _f32),
        pltpu.SemaphoreType.DMA,
        pltpu.SemaphoreType.DMA,
    ],
)(_repack_body)


def _tec_body(xs, ys, zs, tab, scal, out,
              xs_v, ys_v, zs_v, scal_v,
              idx_v0, sel_v0, off_v0, rows_v0, out_v0, sem0, semo0,
              idx_v1, sel_v1, off_v1, rows_v1, out_v1, sem1, semo1):
  wid = lax.axis_index("s") * _NC + lax.axis_index("c")
  base = wid * _PPW
  pltpu.sync_copy(xs.at[pl.ds(base, _PPW)], xs_v)
  pltpu.sync_copy(ys.at[pl.ds(base, _PPW)], ys_v)
  pltpu.sync_copy(zs.at[pl.ds(base, _PPW)], zs_v)
  pltpu.sync_copy(scal, scal_v)
  scalv = scal_v[...]
  iota = lax.iota(_i32, 16)
  lvloff = iota * _TBL
  bufs = ((idx_v0, sel_v0, off_v0, rows_v0, out_v0, sem0, semo0),
          (idx_v1, sel_v1, off_v1, rows_v1, out_v1, sem1, semo1))

  def build_chunk(g, buf):
    idx_v, sel_v, off_v = buf[0], buf[1], buf[2]
    cb = (g * _C).astype(_i32)

    def build(_, p):
      pidx = jnp.broadcast_to(cb + p, (16,))
      x = plsc.load_gather(xs_v, [pidx])
      y = plsc.load_gather(ys_v, [pidx])
      z = plsc.load_gather(zs_v, [pidx])
      sx = x * scalv
      sy = y * scalv
      sz = z * scalv
      fxi = sx.astype(_i32)
      fyi = sy.astype(_i32)
      fzi = sz.astype(_i32)
      off_v[p, pl.ds(0, 16)] = sx - fxi.astype(_f32)
      off_v[p, pl.ds(16, 16)] = sy - fyi.astype(_f32)
      off_v[p, pl.ds(32, 16)] = sz - fzi.astype(_f32)
      tyf = fyi * _P1
      tzf = fzi * _P2
      tyc = tyf + _P1
      tzc = tzf + _P2
      txf = fxi
      txc = fxi + 1
      acc = txc ^ tyc
      acf = txc ^ tyf
      aff = txf ^ tyf
      afc = txf ^ tyc
      hs = (acc ^ tzc, acf ^ tzc, aff ^ tzc, afc ^ tzc,
            acc ^ tzf, acf ^ tzf, aff ^ tzf, afc ^ tzf)
      pb = p * 128
      for c in range(8):
        h = (hs[c] & _MASK) + lvloff
        idx_v[pl.ds(pb + c * 16, 16)] = h >> 2
        sel_v[pl.ds(pb + c * 16, 16)] = (h << 1) & 6
      return p + 1

    lax.fori_loop(np.int32(0), np.int32(_C), build, _i32(0), unroll=2)
    pltpu.async_copy(tab.at[buf[0]], buf[3], buf[5])

  def drain_chunk(g, buf):
    idx_v, sel_v, off_v, rows_v, out_v, sem, semo = buf
    pltpu.make_async_copy(tab.at[idx_v], rows_v, sem).wait()
    cb = (g * _C).astype(_i32)

    @pl.when(g >= 2)
    def _():
      pltpu.make_async_copy(out_v, out.at[pl.ds(base + cb, _C)], semo).wait()

    def interp(_, p):
      o0 = off_v[p, pl.ds(0, 16)]
      o1 = off_v[p, pl.ds(16, 16)]
      o2 = off_v[p, pl.ds(32, 16)]
      pb = p * 128
      f = []
      for c in range(8):
        rvec = pb + c * 16 + iota
        s0 = sel_v[pl.ds(pb + c * 16, 16)]
        f.append((plsc.load_gather(rows_v, [rvec, s0]),
                  plsc.load_gather(rows_v, [rvec, s0 + 1])))

      def lerp(a, b, o):
        return b + (a - b) * o

      prow = jnp.broadcast_to(p, (16,))
      for k in range(2):
        f03 = lerp(f[0][k], f[3][k], o0)
        f12 = lerp(f[1][k], f[2][k], o0)
        f56 = lerp(f[5][k], f[6][k], o0)
        f47 = lerp(f[4][k], f[7][k], o0)
        f0312 = lerp(f03, f12, o1)
        f4756 = lerp(f47, f56, o1)
        plsc.store_scatter(out_v, [prow, 2 * iota + k], lerp(f0312, f4756, o2))
      return p + 1

    lax.fori_loop(np.int32(0), np.int32(_C), interp, _i32(0), unroll=2)
    pltpu.async_copy(out_v, out.at[pl.ds(base + cb, _C)], semo)

  npair = _NCHUNK // 2
  build_chunk(_i32(0), bufs[0])

  def pair(g2, carry):
    ga = (g2 * 2).astype(_i32)
    build_chunk(ga + 1, bufs[1])
    drain_chunk(ga, bufs[0])

    @pl.when(g2 < npair - 1)
    def _():
      build_chunk(ga + 2, bufs[0])

    drain_chunk(ga + 1, bufs[1])
    return carry

  lax.fori_loop(_i32(0), _i32(npair), pair, _i32(0))
  last = _i32(_NCHUNK - 2).astype(_i32) * _C
  pltpu.make_async_copy(out_v0, out.at[pl.ds(base + last, _C)], semo0).wait()
  pltpu.make_async_copy(out_v1, out.at[pl.ds(base + last, _C)], semo1).wait()


_encode = functools.partial(
    pl.kernel,
    out_type=jax.ShapeDtypeStruct((_N, 2 * _NUM_LEVELS), jnp.float32),
    mesh=plsc.VectorSubcoreMesh(core_axis_name="c", subcore_axis_name="s"),
    compiler_params=_COMPILER_PARAMS,
    scratch_types=[
        pltpu.VMEM((_PPW,), _f32),
        pltpu.VMEM((_PPW,), _f32),
        pltpu.VMEM((_PPW,), _f32),
        pltpu.VMEM((_NUM_LEVELS,), _f32),
        pltpu.VMEM((_IPC,), _i32),
        pltpu.VMEM((_IPC,), _i32),
        pltpu.VMEM((_C, 48), _f32),
        pltpu.VMEM((_IPC, 8), _f32),
        pltpu.VMEM((_C, 2 * _NUM_LEVELS), _f32),
        pltpu.SemaphoreType.DMA,
        pltpu.SemaphoreType.DMA,
        pltpu.VMEM((_IPC,), _i32),
        pltpu.VMEM((_IPC,), _i32),
        pltpu.VMEM((_C, 48), _f32),
        pltpu.VMEM((_IPC, 8), _f32),
        pltpu.VMEM((_C, 2 * _NUM_LEVELS), _f32),
        pltpu.SemaphoreType.DMA,
        pltpu.SemaphoreType.DMA,
    ],
)(_tec_body)


def kernel(in_tensor, hash_table):
  x = in_tensor.astype(jnp.float32)
  raw = (hash_table.astype(jnp.float32)
         .reshape(65536, 128, 2).transpose(0, 2, 1).reshape(_TW))
  tab = _repack(raw).reshape(_TW // 8, 8)
  levels = np.arange(_NUM_LEVELS)
  growth = np.exp((np.log(4096.0) - np.log(16.0)) / (_NUM_LEVELS - 1))
  scal = jnp.asarray(np.floor(16.0 * growth ** levels), dtype=jnp.float32)
  return _encode(x[:, 0], x[:, 1], x[:, 2], tab, scal)
